# Initial kernel scaffold; baseline (speedup 1.0000x reference)
import jax
import jax.numpy as jnp
from jax.experimental import pallas as pl


def kernel(x, edge_index, message, W1, b1, W2, b2, W3, gamma, beta):
    raise NotImplementedError("write your pallas kernel here")


# speedup vs baseline: 5.2816x; 5.2816x over previous
import functools

import jax
import jax.numpy as jnp
from jax import lax
from jax.experimental import pallas as pl
from jax.experimental.pallas import tpu as pltpu
from jax.experimental.pallas import tpu_sc as plsc

NC = 2
NS = 16
SUB = 128
UNIT = 512
SPU = UNIT // SUB


def _segment_sum_sc(dest2d, message, zeros):
    n = zeros.shape[0]
    d_msg = message.shape[1]
    e = message.shape[0]
    num_units = e // UNIT
    rows_per_sub = ((n + NS * 8 - 1) // (NS * 8)) * 8

    mesh = plsc.VectorSubcoreMesh(core_axis_name="c", subcore_axis_name="s")

    @functools.partial(
        pl.kernel,
        mesh=mesh,
        compiler_params=pltpu.CompilerParams(use_tc_tiling_on_sc=False),
        out_type=jax.ShapeDtypeStruct((NC, n, d_msg), jnp.float32),
        scratch_types=[
            pltpu.VMEM((UNIT, d_msg), jnp.float32),
            pltpu.VMEM((SPU, SUB), jnp.int32),
            pltpu.VMEM_SHARED((n, d_msg), jnp.float32),
        ],
    )
    def seg_sum(dest_hbm, msg_hbm, zeros_hbm, out_hbm, msg_v, idx_v, acc):
        c = lax.axis_index("c")
        s = lax.axis_index("s")
        wid = c * NS + s

        row0 = pl.multiple_of(
            jnp.minimum(s * rows_per_sub, n - rows_per_sub), 8)
        pltpu.sync_copy(zeros_hbm.at[pl.ds(row0, rows_per_sub)],
                        acc.at[pl.ds(row0, rows_per_sub)])
        plsc.subcore_barrier()

        base = num_units // (NC * NS)
        rem = num_units % (NC * NS)
        extra = jnp.where(wid < rem, 1, 0)
        start = wid * base + jnp.minimum(wid, rem)
        count = base + extra

        def body(i, _):
            u = start + i
            pltpu.sync_copy(dest_hbm.at[pl.ds(u * SPU, SPU)], idx_v)
            pltpu.sync_copy(msg_hbm.at[pl.ds(u * UNIT, UNIT)], msg_v)
            for j in range(SPU):
                pltpu.sync_copy(msg_v.at[pl.ds(j * SUB, SUB)],
                                acc.at[idx_v.at[j]], add=True)
            return 0

        lax.fori_loop(0, count, body, 0)

        plsc.subcore_barrier()
        pltpu.sync_copy(acc.at[pl.ds(row0, rows_per_sub)],
                        out_hbm.at[c].at[pl.ds(row0, rows_per_sub)])

    return seg_sum(dest2d, message, zeros)


def _mlp_body(x_ref, p0_ref, p1_ref, w1x_ref, w1m_ref, b1_ref, w2_ref,
              b2_ref, w3_ref, g_ref, be_ref, o_ref):
    xb = x_ref[...]
    m = p0_ref[...] + p1_ref[...]
    h = (jnp.dot(xb, w1x_ref[...], preferred_element_type=jnp.float32)
         + jnp.dot(m, w1m_ref[...], preferred_element_type=jnp.float32)
         + b1_ref[...])
    h = jnp.where(h >= 0, h, 0.2 * h)
    h = jnp.dot(h, w2_ref[...], preferred_element_type=jnp.float32) + b2_ref[...]
    h = jnp.where(h >= 0, h, 0.2 * h)
    h = jnp.dot(h, w3_ref[...], preferred_element_type=jnp.float32)
    mu = jnp.mean(h, axis=-1, keepdims=True)
    var = jnp.mean((h - mu) ** 2, axis=-1, keepdims=True)
    o_ref[...] = (h - mu) * lax.rsqrt(var + 1e-5) * g_ref[...] + be_ref[...]


def _mlp_tc(x, p0, p1, W1x, W1m, b1, W2, b2, W3, gamma, beta, block_n):
    n, d_in = x.shape
    d_msg = p0.shape[1]
    d_out = W1x.shape[1]
    grid = (n // block_n,)
    return pl.pallas_call(
        _mlp_body,
        grid=grid,
        in_specs=[
            pl.BlockSpec((block_n, d_in), lambda i: (i, 0)),
            pl.BlockSpec((block_n, d_msg), lambda i: (i, 0)),
            pl.BlockSpec((block_n, d_msg), lambda i: (i, 0)),
            pl.BlockSpec((d_in, d_out), lambda i: (0, 0)),
            pl.BlockSpec((d_msg, d_out), lambda i: (0, 0)),
            pl.BlockSpec((1, d_out), lambda i: (0, 0)),
            pl.BlockSpec((d_out, d_out), lambda i: (0, 0)),
            pl.BlockSpec((1, d_out), lambda i: (0, 0)),
            pl.BlockSpec((d_out, d_out), lambda i: (0, 0)),
            pl.BlockSpec((1, d_out), lambda i: (0, 0)),
            pl.BlockSpec((1, d_out), lambda i: (0, 0)),
        ],
        out_specs=pl.BlockSpec((block_n, d_out), lambda i: (i, 0)),
        out_shape=jax.ShapeDtypeStruct((n, d_out), jnp.float32),
        compiler_params=pltpu.CompilerParams(
            dimension_semantics=("arbitrary",),
        ),
    )(x, p0, p1, W1x, W1m, b1, W2, b2, W3, gamma, beta)


def kernel(x, edge_index, message, W1, b1, W2, b2, W3, gamma, beta):
    n, d_in = x.shape
    e, d_msg = message.shape

    dest2d = edge_index[1].reshape(e // SUB, SUB)
    zeros = jnp.zeros((n, d_msg), dtype=jnp.float32)
    parts = _segment_sum_sc(dest2d, message, zeros)

    W1x = W1[:d_in]
    W1m = W1[d_in:]
    out = _mlp_tc(x, parts[0], parts[1], W1x, W1m, b1.reshape(1, -1),
                  W2, b2.reshape(1, -1), W3, gamma.reshape(1, -1),
                  beta.reshape(1, -1), block_n=2000)
    return out
